# baseline (speedup 1.0000x reference)
import functools

import jax
import jax.numpy as jnp
from jax import lax
from jax.experimental import pallas as pl
from jax.experimental.pallas import tpu as pltpu
from jax.experimental.pallas import tpu_sc as plsc

_TILE = 128
_LSUB = 512
_NWORK = 32
_CHUNK = 64
_GATE_ROWS = 1024


def _gate_body(x_ref, gw_ref, gb_ref, i0_ref, i1_ref, s0_ref, s1_ref):
    logits = jnp.dot(x_ref[...], gw_ref[...],
                     preferred_element_type=jnp.float32) + gb_ref[...]
    r, e = logits.shape
    col = lax.broadcasted_iota(jnp.int32, (r, e), 1)
    m1 = jnp.max(logits, axis=1, keepdims=True)
    i1 = jnp.min(jnp.where(logits == m1, col, e), axis=1, keepdims=True)
    masked = jnp.where(col == i1, -jnp.inf, logits)
    m2 = jnp.max(masked, axis=1, keepdims=True)
    i2 = jnp.min(jnp.where(masked == m2, col, e), axis=1, keepdims=True)
    e2 = jnp.exp(m2 - m1)
    denom = 1.0 + e2
    i0_ref[...] = i1
    i1_ref[...] = i2
    s0_ref[...] = 1.0 / denom
    s1_ref[...] = e2 / denom


def _gate_call(moe_inp, gate_w, gate_b):
    n, d = moe_inp.shape
    e = gate_w.shape[1]
    r = _GATE_ROWS
    outs = pl.pallas_call(
        _gate_body,
        grid=(n // r,),
        in_specs=[
            pl.BlockSpec((r, d), lambda i: (i, 0)),
            pl.BlockSpec((d, e), lambda i: (0, 0)),
            pl.BlockSpec((1, e), lambda i: (0, 0)),
        ],
        out_specs=[pl.BlockSpec((r, 1), lambda i: (i, 0))] * 4,
        out_shape=[
            jax.ShapeDtypeStruct((n, 1), jnp.int32),
            jax.ShapeDtypeStruct((n, 1), jnp.int32),
            jax.ShapeDtypeStruct((n, 1), jnp.float32),
            jax.ShapeDtypeStruct((n, 1), jnp.float32),
        ],
    )(moe_inp, gate_w, gate_b.reshape(1, e))
    return outs


def _make_meta_body(n_exp, npad):
    t = _TILE
    nt = npad // t

    def _meta_body(ef_ref, dst_ref, te_ref):
        l, c = ef_ref.shape
        lane_e = lax.broadcasted_iota(jnp.int32, (l, n_exp), 1)
        r_io = lax.broadcasted_iota(jnp.int32, (l, l), 0)
        c_io = lax.broadcasted_iota(jnp.int32, (l, l), 1)
        tril_s = (r_io > c_io).astype(jnp.float32)

        counts = jnp.zeros((1, n_exp), jnp.float32)
        for j in range(c):
            oh = (ef_ref[:, j:j + 1] == lane_e).astype(jnp.float32)
            counts = counts + jnp.sum(oh, axis=0, keepdims=True)

        caps = jnp.floor((counts + (t - 1)) * (1.0 / t)) * t
        er = lax.broadcasted_iota(jnp.int32, (n_exp, n_exp), 0)
        ec = lax.broadcasted_iota(jnp.int32, (n_exp, n_exp), 1)
        sut = (er < ec).astype(jnp.float32)
        off = jnp.dot(caps, sut, preferred_element_type=jnp.float32)

        carry = jnp.zeros((1, n_exp), jnp.float32)
        for j in range(c):
            oh = (ef_ref[:, j:j + 1] == lane_e).astype(jnp.float32)
            r_c = jnp.dot(tril_s, oh, preferred_element_type=jnp.float32)
            dstv = jnp.sum(oh * (off + carry + r_c), axis=1, keepdims=True)
            dst_ref[:, j:j + 1] = dstv.astype(jnp.int32)
            carry = carry + jnp.sum(oh, axis=0, keepdims=True)

        lane_idx = lax.broadcasted_iota(jnp.int32, (1, n_exp), 1)
        end = jnp.where(lane_idx == n_exp - 1, float(npad), off + caps)
        tstart = lax.broadcasted_iota(
            jnp.int32, (nt, n_exp), 0).astype(jnp.float32) * t
        ind = (tstart >= off) & (tstart < end)
        te_f = lax.broadcasted_iota(
            jnp.int32, (nt, n_exp), 1).astype(jnp.float32)
        te = jnp.sum(jnp.where(ind, te_f, 0.0), axis=1, keepdims=True)
        te_ref[...] = te.astype(jnp.int32)

    return _meta_body


def _meta_call(ef_t, n_exp, npad):
    l, c = ef_t.shape
    nt = npad // _TILE
    return pl.pallas_call(
        _make_meta_body(n_exp, npad),
        out_shape=(
            jax.ShapeDtypeStruct((l, c), jnp.int32),
            jax.ShapeDtypeStruct((nt, 1), jnp.int32),
        ),
    )(ef_t)


def _sc_mesh():
    return plsc.VectorSubcoreMesh(core_axis_name="c", subcore_axis_name="s")


def _make_scatter(n, d, npad):
    per_w = n // _NWORK
    nch = per_w // _CHUNK

    @functools.partial(
        pl.kernel,
        mesh=_sc_mesh(),
        out_type=(
            jax.ShapeDtypeStruct((npad, d), jnp.float32),
            jax.ShapeDtypeStruct((npad,), jnp.float32),
        ),
        scratch_types=(
            pltpu.VMEM((_CHUNK, d), jnp.float32),
            pltpu.VMEM((_CHUNK,), jnp.int32),
            pltpu.VMEM((_CHUNK,), jnp.int32),
            pltpu.VMEM((_CHUNK,), jnp.float32),
            pltpu.VMEM((_CHUNK,), jnp.float32),
            pltpu.SemaphoreType.DMA,
        ),
    )
    def scat(moe_hbm, d0_hbm, d1_hbm, s0_hbm, s1_hbm, xs_hbm, ws_hbm,
             rows_v, i0_v, i1_v, v0_v, v1_v, sem):
        wid = lax.axis_index("s") * 2 + lax.axis_index("c")
        for ch in range(nch):
            base = wid * per_w + ch * _CHUNK
            pltpu.sync_copy(moe_hbm.at[pl.ds(base, _CHUNK)], rows_v)
            pltpu.sync_copy(d0_hbm.at[pl.ds(base, _CHUNK)], i0_v)
            pltpu.sync_copy(d1_hbm.at[pl.ds(base, _CHUNK)], i1_v)
            pltpu.sync_copy(s0_hbm.at[pl.ds(base, _CHUNK)], v0_v)
            pltpu.sync_copy(s1_hbm.at[pl.ds(base, _CHUNK)], v1_v)
            pltpu.async_copy(rows_v, xs_hbm.at[i0_v], sem).wait()
            pltpu.async_copy(rows_v, xs_hbm.at[i1_v], sem).wait()
            pltpu.async_copy(v0_v, ws_hbm.at[i0_v], sem).wait()
            pltpu.async_copy(v1_v, ws_hbm.at[i1_v], sem).wait()

    return scat


def _ffn_body(te_ref, x_ref, w1_ref, b1_ref, w2_ref, b2_ref, ws_ref, y_ref):
    del te_ref
    x16 = x_ref[...].astype(jnp.bfloat16)
    h = jnp.dot(x16, w1_ref[0],
                preferred_element_type=jnp.float32) + b1_ref[0]
    h = jnp.maximum(h, 0.0).astype(jnp.bfloat16)
    y = jnp.dot(h, w2_ref[0], preferred_element_type=jnp.float32) + b2_ref[0]
    y_ref[...] = y * ws_ref[...]


def _ffn_call(te, xs, w1, b1, w2, b2, ws, npad):
    e, d, _ = w1.shape
    t = _TILE
    nt = npad // t
    grid_spec = pltpu.PrefetchScalarGridSpec(
        num_scalar_prefetch=1,
        grid=(nt,),
        in_specs=[
            pl.BlockSpec((t, d), lambda i, te_s: (i, 0)),
            pl.BlockSpec((1, d, d), lambda i, te_s: (te_s[i], 0, 0)),
            pl.BlockSpec((1, 1, d), lambda i, te_s: (te_s[i], 0, 0)),
            pl.BlockSpec((1, d, d), lambda i, te_s: (te_s[i], 0, 0)),
            pl.BlockSpec((1, 1, d), lambda i, te_s: (te_s[i], 0, 0)),
            pl.BlockSpec((t, 1), lambda i, te_s: (i, 0)),
        ],
        out_specs=pl.BlockSpec((t, d), lambda i, te_s: (i, 0)),
    )
    return pl.pallas_call(
        _ffn_body,
        grid_spec=grid_spec,
        out_shape=jax.ShapeDtypeStruct((npad, d), jnp.float32),
    )(te, xs, w1.astype(jnp.bfloat16), b1.reshape(e, 1, d),
      w2.astype(jnp.bfloat16), b2.reshape(e, 1, d), ws.reshape(npad, 1))


def _make_combine(n, d, npad):
    per_w = n // _NWORK
    nch = per_w // _CHUNK
    nvec = d // 16

    @functools.partial(
        pl.kernel,
        mesh=_sc_mesh(),
        out_type=jax.ShapeDtypeStruct((n, d), jnp.float32),
        scratch_types=(
            pltpu.VMEM((_CHUNK, d), jnp.float32),
            pltpu.VMEM((_CHUNK, d), jnp.float32),
            pltpu.VMEM((_CHUNK,), jnp.int32),
            pltpu.VMEM((_CHUNK,), jnp.int32),
            pltpu.SemaphoreType.DMA,
        ),
    )
    def comb(y_hbm, d0_hbm, d1_hbm, out_hbm, b0_v, b1_v, i0_v, i1_v, sem):
        wid = lax.axis_index("s") * 2 + lax.axis_index("c")
        for ch in range(nch):
            base = wid * per_w + ch * _CHUNK
            pltpu.sync_copy(d0_hbm.at[pl.ds(base, _CHUNK)], i0_v)
            pltpu.sync_copy(d1_hbm.at[pl.ds(base, _CHUNK)], i1_v)
            pltpu.async_copy(y_hbm.at[i0_v], b0_v, sem).wait()
            pltpu.async_copy(y_hbm.at[i1_v], b1_v, sem).wait()

            @plsc.parallel_loop(0, _CHUNK * nvec, unroll=8)
            def _add(idx):
                r = idx // nvec
                col = (idx % nvec) * 16
                b0_v[r, pl.ds(col, 16)] = (b0_v[r, pl.ds(col, 16)]
                                           + b1_v[r, pl.ds(col, 16)])

            pltpu.sync_copy(b0_v, out_hbm.at[pl.ds(base, _CHUNK)])

    return comb


def kernel(moe_inp, gate_w, gate_b, w1, b1, w2, b2):
    n, d = moe_inp.shape
    n_exp = w1.shape[0]
    npad = 2 * n + n_exp * _TILE

    i0, i1, s0, s1 = _gate_call(moe_inp, gate_w, gate_b)

    ef = jnp.concatenate([i0, i1], axis=0)
    ef_t = ef.reshape(_LSUB, (2 * n) // _LSUB)
    dst_lc, te2 = _meta_call(ef_t, n_exp, npad)
    dstf = dst_lc.reshape(-1)
    d0 = dstf[:n]
    d1 = dstf[n:]

    xs, ws = _make_scatter(n, d, npad)(
        moe_inp, d0, d1, s0.reshape(n), s1.reshape(n))

    y = _ffn_call(te2.reshape(-1), xs, w1, b1, w2, b2, ws, npad)

    out = _make_combine(n, d, npad)(y, d0, d1)
    return out


# speedup vs baseline: 1.1744x; 1.1744x over previous
import functools

import jax
import jax.numpy as jnp
from jax import lax
from jax.experimental import pallas as pl
from jax.experimental.pallas import tpu as pltpu
from jax.experimental.pallas import tpu_sc as plsc

_TILE = 128
_LSUB = 512
_NWORK = 32
_CHUNK = 64
_GATE_ROWS = 1024


def _gate_body(x_ref, gw_ref, gb_ref, i0_ref, i1_ref, s0_ref, s1_ref):
    logits = jnp.dot(x_ref[...], gw_ref[...],
                     preferred_element_type=jnp.float32) + gb_ref[...]
    r, e = logits.shape
    col = lax.broadcasted_iota(jnp.int32, (r, e), 1)
    m1 = jnp.max(logits, axis=1, keepdims=True)
    i1 = jnp.min(jnp.where(logits == m1, col, e), axis=1, keepdims=True)
    masked = jnp.where(col == i1, -jnp.inf, logits)
    m2 = jnp.max(masked, axis=1, keepdims=True)
    i2 = jnp.min(jnp.where(masked == m2, col, e), axis=1, keepdims=True)
    e2 = jnp.exp(m2 - m1)
    denom = 1.0 + e2
    i0_ref[...] = i1
    i1_ref[...] = i2
    s0_ref[...] = 1.0 / denom
    s1_ref[...] = e2 / denom


def _gate_call(moe_inp, gate_w, gate_b):
    n, d = moe_inp.shape
    e = gate_w.shape[1]
    r = _GATE_ROWS
    outs = pl.pallas_call(
        _gate_body,
        grid=(n // r,),
        in_specs=[
            pl.BlockSpec((r, d), lambda i: (i, 0)),
            pl.BlockSpec((d, e), lambda i: (0, 0)),
            pl.BlockSpec((1, e), lambda i: (0, 0)),
        ],
        out_specs=[pl.BlockSpec((r, 1), lambda i: (i, 0))] * 4,
        out_shape=[
            jax.ShapeDtypeStruct((n, 1), jnp.int32),
            jax.ShapeDtypeStruct((n, 1), jnp.int32),
            jax.ShapeDtypeStruct((n, 1), jnp.float32),
            jax.ShapeDtypeStruct((n, 1), jnp.float32),
        ],
    )(moe_inp, gate_w, gate_b.reshape(1, e))
    return outs


def _make_meta_body(n_exp, npad):
    t = _TILE
    nt = npad // t

    def _meta_body(ef_ref, dst_ref, te_ref):
        l, c = ef_ref.shape
        lane_e = lax.broadcasted_iota(jnp.int32, (l, n_exp), 1)
        r_io = lax.broadcasted_iota(jnp.int32, (l, l), 0)
        c_io = lax.broadcasted_iota(jnp.int32, (l, l), 1)
        tril_s = (r_io > c_io).astype(jnp.float32)

        counts = jnp.zeros((1, n_exp), jnp.float32)
        for j in range(c):
            oh = (ef_ref[:, j:j + 1] == lane_e).astype(jnp.float32)
            counts = counts + jnp.sum(oh, axis=0, keepdims=True)

        caps = jnp.floor((counts + (t - 1)) * (1.0 / t)) * t
        er = lax.broadcasted_iota(jnp.int32, (n_exp, n_exp), 0)
        ec = lax.broadcasted_iota(jnp.int32, (n_exp, n_exp), 1)
        sut = (er < ec).astype(jnp.float32)
        off = jnp.dot(caps, sut, preferred_element_type=jnp.float32)

        carry = jnp.zeros((1, n_exp), jnp.float32)
        for j in range(c):
            oh = (ef_ref[:, j:j + 1] == lane_e).astype(jnp.float32)
            r_c = jnp.dot(tril_s, oh, preferred_element_type=jnp.float32)
            dstv = jnp.sum(oh * (off + carry + r_c), axis=1, keepdims=True)
            dst_ref[:, j:j + 1] = dstv.astype(jnp.int32)
            carry = carry + jnp.sum(oh, axis=0, keepdims=True)

        lane_idx = lax.broadcasted_iota(jnp.int32, (1, n_exp), 1)
        end = jnp.where(lane_idx == n_exp - 1, float(npad), off + caps)
        tstart = lax.broadcasted_iota(
            jnp.int32, (nt, n_exp), 0).astype(jnp.float32) * t
        ind = (tstart >= off) & (tstart < end)
        te_f = lax.broadcasted_iota(
            jnp.int32, (nt, n_exp), 1).astype(jnp.float32)
        te = jnp.sum(jnp.where(ind, te_f, 0.0), axis=1, keepdims=True)
        te_ref[...] = te.astype(jnp.int32)

    return _meta_body


def _meta_call(ef_t, n_exp, npad):
    l, c = ef_t.shape
    nt = npad // _TILE
    return pl.pallas_call(
        _make_meta_body(n_exp, npad),
        out_shape=(
            jax.ShapeDtypeStruct((l, c), jnp.int32),
            jax.ShapeDtypeStruct((nt, 1), jnp.int32),
        ),
    )(ef_t)


def _sc_mesh():
    return plsc.VectorSubcoreMesh(core_axis_name="c", subcore_axis_name="s")


def _make_scatter(n, d, npad):
    per_w = n // _NWORK
    nch = per_w // _CHUNK

    @functools.partial(
        pl.kernel,
        mesh=_sc_mesh(),
        out_type=(
            jax.ShapeDtypeStruct((npad, d), jnp.float32),
            jax.ShapeDtypeStruct((npad,), jnp.float32),
        ),
        scratch_types=(
            pltpu.VMEM((2, _CHUNK, d), jnp.float32),
            pltpu.VMEM((2, _CHUNK), jnp.int32),
            pltpu.VMEM((2, _CHUNK), jnp.int32),
            pltpu.VMEM((2, _CHUNK), jnp.float32),
            pltpu.VMEM((2, _CHUNK), jnp.float32),
            pltpu.SemaphoreType.DMA,
            pltpu.SemaphoreType.DMA,
            pltpu.SemaphoreType.DMA,
            pltpu.SemaphoreType.DMA,
        ),
    )
    def scat(moe_hbm, d0_hbm, d1_hbm, s0_hbm, s1_hbm, xs_hbm, ws_hbm,
             rows_v, i0_v, i1_v, v0_v, v1_v, ld0, ld1, st0, st1):
        wid = lax.axis_index("s") * 2 + lax.axis_index("c")
        lds = (ld0, ld1)
        sts = (st0, st1)
        ld_descs = [None, None]
        st_descs = [None, None]

        def load(ch):
            p = ch & 1
            base = wid * per_w + ch * _CHUNK
            ld_descs[p] = [
                pltpu.async_copy(moe_hbm.at[pl.ds(base, _CHUNK)],
                                 rows_v.at[p], lds[p]),
                pltpu.async_copy(d0_hbm.at[pl.ds(base, _CHUNK)],
                                 i0_v.at[p], lds[p]),
                pltpu.async_copy(d1_hbm.at[pl.ds(base, _CHUNK)],
                                 i1_v.at[p], lds[p]),
                pltpu.async_copy(s0_hbm.at[pl.ds(base, _CHUNK)],
                                 v0_v.at[p], lds[p]),
                pltpu.async_copy(s1_hbm.at[pl.ds(base, _CHUNK)],
                                 v1_v.at[p], lds[p]),
            ]

        def store(ch):
            p = ch & 1
            st_descs[p] = [
                pltpu.async_copy(rows_v.at[p], xs_hbm.at[i0_v.at[p]], sts[p]),
                pltpu.async_copy(rows_v.at[p], xs_hbm.at[i1_v.at[p]], sts[p]),
                pltpu.async_copy(v0_v.at[p], ws_hbm.at[i0_v.at[p]], sts[p]),
                pltpu.async_copy(v1_v.at[p], ws_hbm.at[i1_v.at[p]], sts[p]),
            ]

        load(0)
        for ch in range(nch):
            p = ch & 1
            if ch + 1 < nch:
                if ch >= 1:
                    for dd in st_descs[1 - p]:
                        dd.wait()
                load(ch + 1)
            for dd in ld_descs[p]:
                dd.wait()
            store(ch)
        for dd in st_descs[(nch - 2) & 1]:
            dd.wait()
        for dd in st_descs[(nch - 1) & 1]:
            dd.wait()

    return scat


def _ffn_body(te_ref, x_ref, w1_ref, b1_ref, w2_ref, b2_ref, ws_ref, y_ref):
    del te_ref
    h = jnp.dot(x_ref[...], w1_ref[0],
                preferred_element_type=jnp.float32) + b1_ref[0]
    h = jnp.maximum(h, 0.0)
    y = jnp.dot(h, w2_ref[0], preferred_element_type=jnp.float32) + b2_ref[0]
    y_ref[...] = y * ws_ref[...]


def _ffn_call(te, xs, w1, b1, w2, b2, ws, npad):
    e, d, _ = w1.shape
    t = _TILE
    nt = npad // t
    grid_spec = pltpu.PrefetchScalarGridSpec(
        num_scalar_prefetch=1,
        grid=(nt,),
        in_specs=[
            pl.BlockSpec((t, d), lambda i, te_s: (i, 0)),
            pl.BlockSpec((1, d, d), lambda i, te_s: (te_s[i], 0, 0)),
            pl.BlockSpec((1, 1, d), lambda i, te_s: (te_s[i], 0, 0)),
            pl.BlockSpec((1, d, d), lambda i, te_s: (te_s[i], 0, 0)),
            pl.BlockSpec((1, 1, d), lambda i, te_s: (te_s[i], 0, 0)),
            pl.BlockSpec((t, 1), lambda i, te_s: (i, 0)),
        ],
        out_specs=pl.BlockSpec((t, d), lambda i, te_s: (i, 0)),
    )
    return pl.pallas_call(
        _ffn_body,
        grid_spec=grid_spec,
        out_shape=jax.ShapeDtypeStruct((npad, d), jnp.float32),
    )(te, xs, w1, b1.reshape(e, 1, d), w2, b2.reshape(e, 1, d),
      ws.reshape(npad, 1))


def _make_combine(n, d, npad):
    cch = 32
    per_w = n // _NWORK
    nch = per_w // cch
    nvec = d // 16

    @functools.partial(
        pl.kernel,
        mesh=_sc_mesh(),
        out_type=jax.ShapeDtypeStruct((n, d), jnp.float32),
        scratch_types=(
            pltpu.VMEM((2, cch, d), jnp.float32),
            pltpu.VMEM((2, cch, d), jnp.float32),
            pltpu.VMEM((2, cch), jnp.int32),
            pltpu.VMEM((2, cch), jnp.int32),
            pltpu.SemaphoreType.DMA,
            pltpu.SemaphoreType.DMA,
            pltpu.SemaphoreType.DMA,
            pltpu.SemaphoreType.DMA,
            pltpu.SemaphoreType.DMA,
            pltpu.SemaphoreType.DMA,
        ),
    )
    def comb(y_hbm, d0_hbm, d1_hbm, out_hbm, b0_v, b1_v, i0_v, i1_v,
             li0, li1, gd0, gd1, st0, st1):
        wid = lax.axis_index("s") * 2 + lax.axis_index("c")
        lis = (li0, li1)
        gds = (gd0, gd1)
        sts = (st0, st1)
        li_descs = [None, None]
        gd_descs = [None, None]
        st_descs = [None, None]

        def loadidx(ch):
            p = ch & 1
            base = wid * per_w + ch * cch
            li_descs[p] = [
                pltpu.async_copy(d0_hbm.at[pl.ds(base, cch)],
                                 i0_v.at[p], lis[p]),
                pltpu.async_copy(d1_hbm.at[pl.ds(base, cch)],
                                 i1_v.at[p], lis[p]),
            ]

        def gather(ch):
            p = ch & 1
            for dd in li_descs[p]:
                dd.wait()
            gd_descs[p] = [
                pltpu.async_copy(y_hbm.at[i0_v.at[p]], b0_v.at[p], gds[p]),
                pltpu.async_copy(y_hbm.at[i1_v.at[p]], b1_v.at[p], gds[p]),
            ]

        loadidx(0)
        gather(0)
        for ch in range(nch):
            p = ch & 1
            base = wid * per_w + ch * cch
            if ch + 1 < nch:
                loadidx(ch + 1)
            for dd in gd_descs[p]:
                dd.wait()
            if ch + 1 < nch:
                if ch >= 1:
                    for dd in st_descs[1 - p]:
                        dd.wait()
                gather(ch + 1)

            @plsc.parallel_loop(0, cch * nvec, unroll=8)
            def _add(idx):
                r = idx // nvec
                col = (idx % nvec) * 16
                b0_v[p, r, pl.ds(col, 16)] = (b0_v[p, r, pl.ds(col, 16)]
                                              + b1_v[p, r, pl.ds(col, 16)])

            st_descs[p] = [
                pltpu.async_copy(b0_v.at[p],
                                 out_hbm.at[pl.ds(base, cch)], sts[p]),
            ]
        for dd in st_descs[(nch - 2) & 1]:
            dd.wait()
        for dd in st_descs[(nch - 1) & 1]:
            dd.wait()

    return comb


def kernel(moe_inp, gate_w, gate_b, w1, b1, w2, b2):
    n, d = moe_inp.shape
    n_exp = w1.shape[0]
    npad = 2 * n + n_exp * _TILE

    i0, i1, s0, s1 = _gate_call(moe_inp, gate_w, gate_b)

    ef = jnp.concatenate([i0, i1], axis=0)
    ef_t = ef.reshape(_LSUB, (2 * n) // _LSUB)
    dst_lc, te2 = _meta_call(ef_t, n_exp, npad)
    dstf = dst_lc.reshape(-1)
    d0 = dstf[:n]
    d1 = dstf[n:]

    xs, ws = _make_scatter(n, d, npad)(
        moe_inp, d0, d1, s0.reshape(n), s1.reshape(n))

    y = _ffn_call(te2.reshape(-1), xs, w1, b1, w2, b2, ws, npad)

    out = _make_combine(n, d, npad)(y, d0, d1)
    return out


# speedup vs baseline: 1.2235x; 1.0418x over previous
import functools

import jax
import jax.numpy as jnp
from jax import lax
from jax.experimental import pallas as pl
from jax.experimental.pallas import tpu as pltpu
from jax.experimental.pallas import tpu_sc as plsc

_TILE = 128
_LSUB = 512
_NWORK = 32
_CHUNK = 64
_GATE_ROWS = 1024


def _gate_body(x_ref, gw_ref, gb_ref, i0_ref, i1_ref, s0_ref, s1_ref,
               x16_ref):
    x = x_ref[...]
    half = x.shape[1] // 2
    xa = lax.bitcast_convert_type(x[:, :half].astype(jnp.bfloat16),
                                  jnp.uint16).astype(jnp.uint32)
    xb = lax.bitcast_convert_type(x[:, half:].astype(jnp.bfloat16),
                                  jnp.uint16).astype(jnp.uint32)
    x16_ref[...] = lax.bitcast_convert_type(xa | (xb << 16), jnp.int32)
    logits = jnp.dot(x, gw_ref[...],
                     preferred_element_type=jnp.float32) + gb_ref[...]
    r, e = logits.shape
    col = lax.broadcasted_iota(jnp.int32, (r, e), 1)
    m1 = jnp.max(logits, axis=1, keepdims=True)
    i1 = jnp.min(jnp.where(logits == m1, col, e), axis=1, keepdims=True)
    masked = jnp.where(col == i1, -jnp.inf, logits)
    m2 = jnp.max(masked, axis=1, keepdims=True)
    i2 = jnp.min(jnp.where(masked == m2, col, e), axis=1, keepdims=True)
    e2 = jnp.exp(m2 - m1)
    denom = 1.0 + e2
    i0_ref[...] = i1
    i1_ref[...] = i2
    s0_ref[...] = 1.0 / denom
    s1_ref[...] = e2 / denom


def _gate_call(moe_inp, gate_w, gate_b):
    n, d = moe_inp.shape
    e = gate_w.shape[1]
    r = _GATE_ROWS
    outs = pl.pallas_call(
        _gate_body,
        grid=(n // r,),
        in_specs=[
            pl.BlockSpec((r, d), lambda i: (i, 0)),
            pl.BlockSpec((d, e), lambda i: (0, 0)),
            pl.BlockSpec((1, e), lambda i: (0, 0)),
        ],
        out_specs=[pl.BlockSpec((r, 1), lambda i: (i, 0))] * 4
        + [pl.BlockSpec((r, d // 2), lambda i: (i, 0))],
        out_shape=[
            jax.ShapeDtypeStruct((n, 1), jnp.int32),
            jax.ShapeDtypeStruct((n, 1), jnp.int32),
            jax.ShapeDtypeStruct((n, 1), jnp.float32),
            jax.ShapeDtypeStruct((n, 1), jnp.float32),
            jax.ShapeDtypeStruct((n, d // 2), jnp.int32),
        ],
    )(moe_inp, gate_w, gate_b.reshape(1, e))
    return outs


def _make_meta_body(n_exp, npad):
    t = _TILE
    nt = npad // t

    def _meta_body(ef_ref, dst_ref, te_ref):
        l, c = ef_ref.shape
        lane_e = lax.broadcasted_iota(jnp.int32, (l, n_exp), 1)
        r_io = lax.broadcasted_iota(jnp.int32, (l, l), 0)
        c_io = lax.broadcasted_iota(jnp.int32, (l, l), 1)
        tril_s = (r_io > c_io).astype(jnp.float32)

        counts = jnp.zeros((1, n_exp), jnp.float32)
        for j in range(c):
            oh = (ef_ref[:, j:j + 1] == lane_e).astype(jnp.float32)
            counts = counts + jnp.sum(oh, axis=0, keepdims=True)

        caps = jnp.floor((counts + (t - 1)) * (1.0 / t)) * t
        er = lax.broadcasted_iota(jnp.int32, (n_exp, n_exp), 0)
        ec = lax.broadcasted_iota(jnp.int32, (n_exp, n_exp), 1)
        sut = (er < ec).astype(jnp.float32)
        off = jnp.dot(caps, sut, preferred_element_type=jnp.float32)

        carry = jnp.zeros((1, n_exp), jnp.float32)
        for j in range(c):
            oh = (ef_ref[:, j:j + 1] == lane_e).astype(jnp.float32)
            r_c = jnp.dot(tril_s, oh, preferred_element_type=jnp.float32)
            dstv = jnp.sum(oh * (off + carry + r_c), axis=1, keepdims=True)
            dst_ref[:, j:j + 1] = dstv.astype(jnp.int32)
            carry = carry + jnp.sum(oh, axis=0, keepdims=True)

        lane_idx = lax.broadcasted_iota(jnp.int32, (1, n_exp), 1)
        end = jnp.where(lane_idx == n_exp - 1, float(npad), off + caps)
        tstart = lax.broadcasted_iota(
            jnp.int32, (nt, n_exp), 0).astype(jnp.float32) * t
        ind = (tstart >= off) & (tstart < end)
        te_f = lax.broadcasted_iota(
            jnp.int32, (nt, n_exp), 1).astype(jnp.float32)
        te = jnp.sum(jnp.where(ind, te_f, 0.0), axis=1, keepdims=True)
        te_ref[...] = te.astype(jnp.int32)

    return _meta_body


def _meta_call(ef_t, n_exp, npad):
    l, c = ef_t.shape
    nt = npad // _TILE
    return pl.pallas_call(
        _make_meta_body(n_exp, npad),
        out_shape=(
            jax.ShapeDtypeStruct((l, c), jnp.int32),
            jax.ShapeDtypeStruct((nt, 1), jnp.int32),
        ),
    )(ef_t)


def _sc_mesh():
    return plsc.VectorSubcoreMesh(core_axis_name="c", subcore_axis_name="s")


def _make_scatter(n, dpack, npad):
    per_w = n // _NWORK
    nch = per_w // _CHUNK

    @functools.partial(
        pl.kernel,
        mesh=_sc_mesh(),
        out_type=(
            jax.ShapeDtypeStruct((npad, dpack), jnp.int32),
            jax.ShapeDtypeStruct((npad,), jnp.float32),
        ),
        scratch_types=(
            pltpu.VMEM((2, _CHUNK, dpack), jnp.int32),
            pltpu.VMEM((2, _CHUNK), jnp.int32),
            pltpu.VMEM((2, _CHUNK), jnp.int32),
            pltpu.VMEM((2, _CHUNK), jnp.float32),
            pltpu.VMEM((2, _CHUNK), jnp.float32),
            pltpu.SemaphoreType.DMA,
            pltpu.SemaphoreType.DMA,
            pltpu.SemaphoreType.DMA,
            pltpu.SemaphoreType.DMA,
        ),
    )
    def scat(moe_hbm, d0_hbm, d1_hbm, s0_hbm, s1_hbm, xs_hbm, ws_hbm,
             rows_v, i0_v, i1_v, v0_v, v1_v, ld0, ld1, st0, st1):
        wid = lax.axis_index("s") * 2 + lax.axis_index("c")
        lds = (ld0, ld1)
        sts = (st0, st1)
        ld_descs = [None, None]
        st_descs = [None, None]

        def load(ch):
            p = ch & 1
            base = wid * per_w + ch * _CHUNK
            ld_descs[p] = [
                pltpu.async_copy(moe_hbm.at[pl.ds(base, _CHUNK)],
                                 rows_v.at[p], lds[p]),
                pltpu.async_copy(d0_hbm.at[pl.ds(base, _CHUNK)],
                                 i0_v.at[p], lds[p]),
                pltpu.async_copy(d1_hbm.at[pl.ds(base, _CHUNK)],
                                 i1_v.at[p], lds[p]),
                pltpu.async_copy(s0_hbm.at[pl.ds(base, _CHUNK)],
                                 v0_v.at[p], lds[p]),
                pltpu.async_copy(s1_hbm.at[pl.ds(base, _CHUNK)],
                                 v1_v.at[p], lds[p]),
            ]

        def store(ch):
            p = ch & 1
            st_descs[p] = [
                pltpu.async_copy(rows_v.at[p], xs_hbm.at[i0_v.at[p]], sts[p]),
                pltpu.async_copy(rows_v.at[p], xs_hbm.at[i1_v.at[p]], sts[p]),
                pltpu.async_copy(v0_v.at[p], ws_hbm.at[i0_v.at[p]], sts[p]),
                pltpu.async_copy(v1_v.at[p], ws_hbm.at[i1_v.at[p]], sts[p]),
            ]

        load(0)
        for ch in range(nch):
            p = ch & 1
            if ch + 1 < nch:
                if ch >= 1:
                    for dd in st_descs[1 - p]:
                        dd.wait()
                load(ch + 1)
            for dd in ld_descs[p]:
                dd.wait()
            store(ch)
        for dd in st_descs[(nch - 2) & 1]:
            dd.wait()
        for dd in st_descs[(nch - 1) & 1]:
            dd.wait()

    return scat


def _ffn_body(te_ref, x_ref, w1_ref, b1_ref, w2_ref, b2_ref, ws_ref, y_ref):
    del te_ref
    xi = x_ref[...]
    lo = lax.bitcast_convert_type(xi << 16, jnp.float32)
    hi = lax.bitcast_convert_type(xi & jnp.int32(-65536), jnp.float32)
    x = jnp.concatenate([lo, hi], axis=1)
    h = jnp.dot(x, w1_ref[0],
                preferred_element_type=jnp.float32) + b1_ref[0]
    h = jnp.maximum(h, 0.0)
    y = jnp.dot(h, w2_ref[0], preferred_element_type=jnp.float32) + b2_ref[0]
    y_ref[...] = y * ws_ref[...]


def _ffn_call(te, xs, w1, b1, w2, b2, ws, npad):
    e, d, _ = w1.shape
    t = _TILE
    nt = npad // t
    grid_spec = pltpu.PrefetchScalarGridSpec(
        num_scalar_prefetch=1,
        grid=(nt,),
        in_specs=[
            pl.BlockSpec((t, d // 2), lambda i, te_s: (i, 0)),
            pl.BlockSpec((1, d, d), lambda i, te_s: (te_s[i], 0, 0)),
            pl.BlockSpec((1, 1, d), lambda i, te_s: (te_s[i], 0, 0)),
            pl.BlockSpec((1, d, d), lambda i, te_s: (te_s[i], 0, 0)),
            pl.BlockSpec((1, 1, d), lambda i, te_s: (te_s[i], 0, 0)),
            pl.BlockSpec((t, 1), lambda i, te_s: (i, 0)),
        ],
        out_specs=pl.BlockSpec((t, d), lambda i, te_s: (i, 0)),
    )
    return pl.pallas_call(
        _ffn_body,
        grid_spec=grid_spec,
        out_shape=jax.ShapeDtypeStruct((npad, d), jnp.float32),
    )(te, xs, w1, b1.reshape(e, 1, d), w2, b2.reshape(e, 1, d),
      ws.reshape(npad, 1))


def _make_combine(n, d, npad):
    cch = 32
    per_w = n // _NWORK
    nch = per_w // cch
    nvec = d // 16

    @functools.partial(
        pl.kernel,
        mesh=_sc_mesh(),
        out_type=jax.ShapeDtypeStruct((n, d), jnp.float32),
        scratch_types=(
            pltpu.VMEM((2, cch, d), jnp.float32),
            pltpu.VMEM((2, cch, d), jnp.float32),
            pltpu.VMEM((2, cch), jnp.int32),
            pltpu.VMEM((2, cch), jnp.int32),
            pltpu.SemaphoreType.DMA,
            pltpu.SemaphoreType.DMA,
            pltpu.SemaphoreType.DMA,
            pltpu.SemaphoreType.DMA,
            pltpu.SemaphoreType.DMA,
            pltpu.SemaphoreType.DMA,
        ),
    )
    def comb(y_hbm, d0_hbm, d1_hbm, out_hbm, b0_v, b1_v, i0_v, i1_v,
             li0, li1, gd0, gd1, st0, st1):
        wid = lax.axis_index("s") * 2 + lax.axis_index("c")
        lis = (li0, li1)
        gds = (gd0, gd1)
        sts = (st0, st1)
        li_descs = [None, None]
        gd_descs = [None, None]
        st_descs = [None, None]

        def loadidx(ch):
            p = ch & 1
            base = wid * per_w + ch * cch
            li_descs[p] = [
                pltpu.async_copy(d0_hbm.at[pl.ds(base, cch)],
                                 i0_v.at[p], lis[p]),
                pltpu.async_copy(d1_hbm.at[pl.ds(base, cch)],
                                 i1_v.at[p], lis[p]),
            ]

        def gather(ch):
            p = ch & 1
            for dd in li_descs[p]:
                dd.wait()
            gd_descs[p] = [
                pltpu.async_copy(y_hbm.at[i0_v.at[p]], b0_v.at[p], gds[p]),
                pltpu.async_copy(y_hbm.at[i1_v.at[p]], b1_v.at[p], gds[p]),
            ]

        loadidx(0)
        gather(0)
        for ch in range(nch):
            p = ch & 1
            base = wid * per_w + ch * cch
            if ch + 1 < nch:
                loadidx(ch + 1)
            for dd in gd_descs[p]:
                dd.wait()
            if ch + 1 < nch:
                if ch >= 1:
                    for dd in st_descs[1 - p]:
                        dd.wait()
                gather(ch + 1)

            @plsc.parallel_loop(0, cch * nvec, unroll=8)
            def _add(idx):
                r = idx // nvec
                col = (idx % nvec) * 16
                b0_v[p, r, pl.ds(col, 16)] = (b0_v[p, r, pl.ds(col, 16)]
                                              + b1_v[p, r, pl.ds(col, 16)])

            st_descs[p] = [
                pltpu.async_copy(b0_v.at[p],
                                 out_hbm.at[pl.ds(base, cch)], sts[p]),
            ]
        for dd in st_descs[(nch - 2) & 1]:
            dd.wait()
        for dd in st_descs[(nch - 1) & 1]:
            dd.wait()

    return comb


def kernel(moe_inp, gate_w, gate_b, w1, b1, w2, b2):
    n, d = moe_inp.shape
    n_exp = w1.shape[0]
    npad = 2 * n + n_exp * _TILE

    i0, i1, s0, s1, x16 = _gate_call(moe_inp, gate_w, gate_b)

    ef = jnp.concatenate([i0, i1], axis=0)
    ef_t = ef.reshape(_LSUB, (2 * n) // _LSUB)
    dst_lc, te2 = _meta_call(ef_t, n_exp, npad)
    dstf = dst_lc.reshape(-1)
    d0 = dstf[:n]
    d1 = dstf[n:]

    xs, ws = _make_scatter(n, d // 2, npad)(
        x16, d0, d1, s0.reshape(n), s1.reshape(n))

    y = _ffn_call(te2.reshape(-1), xs, w1, b1, w2, b2, ws, npad)

    out = _make_combine(n, d, npad)(y, d0, d1)
    return out


# speedup vs baseline: 1.2850x; 1.0503x over previous
import functools

import jax
import jax.numpy as jnp
from jax import lax
from jax.experimental import pallas as pl
from jax.experimental.pallas import tpu as pltpu
from jax.experimental.pallas import tpu_sc as plsc

_TILE = 128
_LSUB = 512
_NWORK = 32
_CHUNK = 64
_GATE_ROWS = 1024


def _gate_body(x_ref, gw_ref, gb_ref, i0_ref, i1_ref, s0_ref, s1_ref,
               x16_ref):
    x = x_ref[...]
    half = x.shape[1] // 2
    xa = lax.bitcast_convert_type(x[:, :half].astype(jnp.bfloat16),
                                  jnp.uint16).astype(jnp.uint32)
    xb = lax.bitcast_convert_type(x[:, half:].astype(jnp.bfloat16),
                                  jnp.uint16).astype(jnp.uint32)
    x16_ref[...] = lax.bitcast_convert_type(xa | (xb << 16), jnp.int32)
    logits = jnp.dot(x, gw_ref[...],
                     preferred_element_type=jnp.float32) + gb_ref[...]
    r, e = logits.shape
    col = lax.broadcasted_iota(jnp.int32, (r, e), 1)
    m1 = jnp.max(logits, axis=1, keepdims=True)
    i1 = jnp.min(jnp.where(logits == m1, col, e), axis=1, keepdims=True)
    masked = jnp.where(col == i1, -jnp.inf, logits)
    m2 = jnp.max(masked, axis=1, keepdims=True)
    i2 = jnp.min(jnp.where(masked == m2, col, e), axis=1, keepdims=True)
    e2 = jnp.exp(m2 - m1)
    denom = 1.0 + e2
    i0_ref[...] = i1
    i1_ref[...] = i2
    s0_ref[...] = 1.0 / denom
    s1_ref[...] = e2 / denom


def _gate_call(moe_inp, gate_w, gate_b):
    n, d = moe_inp.shape
    e = gate_w.shape[1]
    r = _GATE_ROWS
    outs = pl.pallas_call(
        _gate_body,
        grid=(n // r,),
        in_specs=[
            pl.BlockSpec((r, d), lambda i: (i, 0)),
            pl.BlockSpec((d, e), lambda i: (0, 0)),
            pl.BlockSpec((1, e), lambda i: (0, 0)),
        ],
        out_specs=[pl.BlockSpec((r, 1), lambda i: (i, 0))] * 4
        + [pl.BlockSpec((r, d // 2), lambda i: (i, 0))],
        out_shape=[
            jax.ShapeDtypeStruct((n, 1), jnp.int32),
            jax.ShapeDtypeStruct((n, 1), jnp.int32),
            jax.ShapeDtypeStruct((n, 1), jnp.float32),
            jax.ShapeDtypeStruct((n, 1), jnp.float32),
            jax.ShapeDtypeStruct((n, d // 2), jnp.int32),
        ],
    )(moe_inp, gate_w, gate_b.reshape(1, e))
    return outs


def _make_meta_body(n_exp, npad):
    t = _TILE
    nt = npad // t

    def _meta_body(ef_ref, dst_ref, te_ref):
        l, c = ef_ref.shape
        lane_e = lax.broadcasted_iota(jnp.int32, (l, n_exp), 1)
        r_io = lax.broadcasted_iota(jnp.int32, (l, l), 0)
        c_io = lax.broadcasted_iota(jnp.int32, (l, l), 1)
        tril_s = (r_io > c_io).astype(jnp.float32)

        counts = jnp.zeros((1, n_exp), jnp.float32)
        for j in range(c):
            oh = (ef_ref[:, j:j + 1] == lane_e).astype(jnp.float32)
            counts = counts + jnp.sum(oh, axis=0, keepdims=True)

        caps = jnp.floor((counts + (t - 1)) * (1.0 / t)) * t
        er = lax.broadcasted_iota(jnp.int32, (n_exp, n_exp), 0)
        ec = lax.broadcasted_iota(jnp.int32, (n_exp, n_exp), 1)
        sut = (er < ec).astype(jnp.float32)
        off = jnp.dot(caps, sut, preferred_element_type=jnp.float32)

        carry = jnp.zeros((1, n_exp), jnp.float32)
        for j in range(c):
            oh = (ef_ref[:, j:j + 1] == lane_e).astype(jnp.float32)
            r_c = jnp.dot(tril_s, oh, preferred_element_type=jnp.float32)
            dstv = jnp.sum(oh * (off + carry + r_c), axis=1, keepdims=True)
            dst_ref[:, j:j + 1] = dstv.astype(jnp.int32)
            carry = carry + jnp.sum(oh, axis=0, keepdims=True)

        lane_idx = lax.broadcasted_iota(jnp.int32, (1, n_exp), 1)
        end = jnp.where(lane_idx == n_exp - 1, float(npad), off + caps)
        tstart = lax.broadcasted_iota(
            jnp.int32, (nt, n_exp), 0).astype(jnp.float32) * t
        ind = (tstart >= off) & (tstart < end)
        te_f = lax.broadcasted_iota(
            jnp.int32, (nt, n_exp), 1).astype(jnp.float32)
        te = jnp.sum(jnp.where(ind, te_f, 0.0), axis=1, keepdims=True)
        te_ref[...] = te.astype(jnp.int32)

    return _meta_body


def _meta_call(ef_t, n_exp, npad):
    l, c = ef_t.shape
    nt = npad // _TILE
    return pl.pallas_call(
        _make_meta_body(n_exp, npad),
        out_shape=(
            jax.ShapeDtypeStruct((l, c), jnp.int32),
            jax.ShapeDtypeStruct((nt, 1), jnp.int32),
        ),
    )(ef_t)


def _sc_mesh():
    return plsc.VectorSubcoreMesh(core_axis_name="c", subcore_axis_name="s")


def _make_scatter(n, dpack, npad):
    per_w = n // _NWORK
    nch = per_w // _CHUNK

    @functools.partial(
        pl.kernel,
        mesh=_sc_mesh(),
        out_type=(
            jax.ShapeDtypeStruct((npad, dpack), jnp.int32),
            jax.ShapeDtypeStruct((npad,), jnp.float32),
        ),
        scratch_types=(
            pltpu.VMEM((2, _CHUNK, dpack), jnp.int32),
            pltpu.VMEM((2, _CHUNK), jnp.int32),
            pltpu.VMEM((2, _CHUNK), jnp.int32),
            pltpu.VMEM((2, _CHUNK), jnp.float32),
            pltpu.VMEM((2, _CHUNK), jnp.float32),
            pltpu.SemaphoreType.DMA,
            pltpu.SemaphoreType.DMA,
            pltpu.SemaphoreType.DMA,
            pltpu.SemaphoreType.DMA,
        ),
    )
    def scat(moe_hbm, d0_hbm, d1_hbm, s0_hbm, s1_hbm, xs_hbm, ws_hbm,
             rows_v, i0_v, i1_v, v0_v, v1_v, ld0, ld1, st0, st1):
        wid = lax.axis_index("s") * 2 + lax.axis_index("c")
        lds = (ld0, ld1)
        sts = (st0, st1)
        ld_descs = [None, None]
        st_descs = [None, None]

        def load(ch):
            p = ch & 1
            base = wid * per_w + ch * _CHUNK
            ld_descs[p] = [
                pltpu.async_copy(moe_hbm.at[pl.ds(base, _CHUNK)],
                                 rows_v.at[p], lds[p]),
                pltpu.async_copy(d0_hbm.at[pl.ds(base, _CHUNK)],
                                 i0_v.at[p], lds[p]),
                pltpu.async_copy(d1_hbm.at[pl.ds(base, _CHUNK)],
                                 i1_v.at[p], lds[p]),
                pltpu.async_copy(s0_hbm.at[pl.ds(base, _CHUNK)],
                                 v0_v.at[p], lds[p]),
                pltpu.async_copy(s1_hbm.at[pl.ds(base, _CHUNK)],
                                 v1_v.at[p], lds[p]),
            ]

        def store(ch):
            p = ch & 1
            st_descs[p] = [
                pltpu.async_copy(rows_v.at[p], xs_hbm.at[i0_v.at[p]], sts[p]),
                pltpu.async_copy(rows_v.at[p], xs_hbm.at[i1_v.at[p]], sts[p]),
                pltpu.async_copy(v0_v.at[p], ws_hbm.at[i0_v.at[p]], sts[p]),
                pltpu.async_copy(v1_v.at[p], ws_hbm.at[i1_v.at[p]], sts[p]),
            ]

        load(0)
        for ch in range(nch):
            p = ch & 1
            if ch + 1 < nch:
                if ch >= 1:
                    for dd in st_descs[1 - p]:
                        dd.wait()
                load(ch + 1)
            for dd in ld_descs[p]:
                dd.wait()
            store(ch)
        for dd in st_descs[(nch - 2) & 1]:
            dd.wait()
        for dd in st_descs[(nch - 1) & 1]:
            dd.wait()

    return scat


def _ffn_body(te_ref, x_ref, w1_ref, b1_ref, w2_ref, b2_ref, ws_ref, y_ref):
    del te_ref
    xi = x_ref[...]
    lo = lax.bitcast_convert_type(xi << 16, jnp.float32)
    hi = lax.bitcast_convert_type(xi & jnp.int32(-65536), jnp.float32)
    x = jnp.concatenate([lo, hi], axis=1)
    h = jnp.dot(x, w1_ref[0],
                preferred_element_type=jnp.float32) + b1_ref[0]
    h = jnp.maximum(h, 0.0)
    y = jnp.dot(h, w2_ref[0], preferred_element_type=jnp.float32) + b2_ref[0]
    y = y * ws_ref[...]
    half = y.shape[1] // 2
    ya = lax.bitcast_convert_type(y[:, :half].astype(jnp.bfloat16),
                                  jnp.uint16).astype(jnp.uint32)
    yb = lax.bitcast_convert_type(y[:, half:].astype(jnp.bfloat16),
                                  jnp.uint16).astype(jnp.uint32)
    y_ref[...] = lax.bitcast_convert_type(ya | (yb << 16), jnp.int32)


def _ffn_call(te, xs, w1, b1, w2, b2, ws, npad):
    e, d, _ = w1.shape
    t = _TILE
    nt = npad // t
    grid_spec = pltpu.PrefetchScalarGridSpec(
        num_scalar_prefetch=1,
        grid=(nt,),
        in_specs=[
            pl.BlockSpec((t, d // 2), lambda i, te_s: (i, 0)),
            pl.BlockSpec((1, d, d), lambda i, te_s: (te_s[i], 0, 0)),
            pl.BlockSpec((1, 1, d), lambda i, te_s: (te_s[i], 0, 0)),
            pl.BlockSpec((1, d, d), lambda i, te_s: (te_s[i], 0, 0)),
            pl.BlockSpec((1, 1, d), lambda i, te_s: (te_s[i], 0, 0)),
            pl.BlockSpec((t, 1), lambda i, te_s: (i, 0)),
        ],
        out_specs=pl.BlockSpec((t, d // 2), lambda i, te_s: (i, 0)),
    )
    return pl.pallas_call(
        _ffn_body,
        grid_spec=grid_spec,
        out_shape=jax.ShapeDtypeStruct((npad, d // 2), jnp.int32),
    )(te, xs, w1, b1.reshape(e, 1, d), w2, b2.reshape(e, 1, d),
      ws.reshape(npad, 1))


def _make_combine(n, d, npad):
    cch = 32
    per_w = n // _NWORK
    nch = per_w // cch
    dpack = d // 2
    nvec = dpack // 16

    @functools.partial(
        pl.kernel,
        mesh=_sc_mesh(),
        out_type=jax.ShapeDtypeStruct((n, d), jnp.float32),
        scratch_types=(
            pltpu.VMEM((2, cch, dpack), jnp.int32),
            pltpu.VMEM((2, cch, dpack), jnp.int32),
            pltpu.VMEM((2, cch, d), jnp.float32),
            pltpu.VMEM((2, cch), jnp.int32),
            pltpu.VMEM((2, cch), jnp.int32),
            pltpu.SemaphoreType.DMA,
            pltpu.SemaphoreType.DMA,
            pltpu.SemaphoreType.DMA,
            pltpu.SemaphoreType.DMA,
            pltpu.SemaphoreType.DMA,
            pltpu.SemaphoreType.DMA,
        ),
    )
    def comb(y_hbm, d0_hbm, d1_hbm, out_hbm, b0_v, b1_v, o_v, i0_v, i1_v,
             li0, li1, gd0, gd1, st0, st1):
        wid = lax.axis_index("s") * 2 + lax.axis_index("c")
        lis = (li0, li1)
        gds = (gd0, gd1)
        sts = (st0, st1)
        li_descs = [None, None]
        gd_descs = [None, None]
        st_descs = [None, None]

        def loadidx(ch):
            p = ch & 1
            base = wid * per_w + ch * cch
            li_descs[p] = [
                pltpu.async_copy(d0_hbm.at[pl.ds(base, cch)],
                                 i0_v.at[p], lis[p]),
                pltpu.async_copy(d1_hbm.at[pl.ds(base, cch)],
                                 i1_v.at[p], lis[p]),
            ]

        def gather(ch):
            p = ch & 1
            for dd in li_descs[p]:
                dd.wait()
            gd_descs[p] = [
                pltpu.async_copy(y_hbm.at[i0_v.at[p]], b0_v.at[p], gds[p]),
                pltpu.async_copy(y_hbm.at[i1_v.at[p]], b1_v.at[p], gds[p]),
            ]

        loadidx(0)
        gather(0)
        for ch in range(nch):
            p = ch & 1
            base = wid * per_w + ch * cch
            if ch + 1 < nch:
                loadidx(ch + 1)
            for dd in gd_descs[p]:
                dd.wait()
            if ch + 1 < nch:
                if ch >= 1:
                    for dd in st_descs[1 - p]:
                        dd.wait()
                gather(ch + 1)

            @plsc.parallel_loop(0, cch * nvec, unroll=8)
            def _add(idx):
                r = idx // nvec
                col = (idx % nvec) * 16
                v0 = b0_v[p, r, pl.ds(col, 16)]
                v1 = b1_v[p, r, pl.ds(col, 16)]
                mask = jnp.int32(-65536)
                lo = (lax.bitcast_convert_type(v0 << 16, jnp.float32)
                      + lax.bitcast_convert_type(v1 << 16, jnp.float32))
                hi = (lax.bitcast_convert_type(v0 & mask, jnp.float32)
                      + lax.bitcast_convert_type(v1 & mask, jnp.float32))
                o_v[p, r, pl.ds(col, 16)] = lo
                o_v[p, r, pl.ds(dpack + col, 16)] = hi

            st_descs[p] = [
                pltpu.async_copy(o_v.at[p],
                                 out_hbm.at[pl.ds(base, cch)], sts[p]),
            ]
        for dd in st_descs[(nch - 2) & 1]:
            dd.wait()
        for dd in st_descs[(nch - 1) & 1]:
            dd.wait()

    return comb


def kernel(moe_inp, gate_w, gate_b, w1, b1, w2, b2):
    n, d = moe_inp.shape
    n_exp = w1.shape[0]
    npad = 2 * n + n_exp * _TILE

    i0, i1, s0, s1, x16 = _gate_call(moe_inp, gate_w, gate_b)

    ef = jnp.concatenate([i0, i1], axis=0)
    ef_t = ef.reshape(_LSUB, (2 * n) // _LSUB)
    dst_lc, te2 = _meta_call(ef_t, n_exp, npad)
    dstf = dst_lc.reshape(-1)
    d0 = dstf[:n]
    d1 = dstf[n:]

    xs, ws = _make_scatter(n, d // 2, npad)(
        x16, d0, d1, s0.reshape(n), s1.reshape(n))

    y = _ffn_call(te2.reshape(-1), xs, w1, b1, w2, b2, ws, npad)

    out = _make_combine(n, d, npad)(y, d0, d1)
    return out


# speedup vs baseline: 1.2863x; 1.0010x over previous
import functools

import jax
import jax.numpy as jnp
from jax import lax
from jax.experimental import pallas as pl
from jax.experimental.pallas import tpu as pltpu
from jax.experimental.pallas import tpu_sc as plsc

_TILE = 128
_LSUB = 512
_NWORK = 32
_CHUNK = 128
_GATE_ROWS = 1024


def _gate_body(x_ref, gw_ref, gb_ref, i0_ref, i1_ref, s0_ref, s1_ref,
               x16_ref):
    x = x_ref[...]
    half = x.shape[1] // 2
    xa = lax.bitcast_convert_type(x[:, :half].astype(jnp.bfloat16),
                                  jnp.uint16).astype(jnp.uint32)
    xb = lax.bitcast_convert_type(x[:, half:].astype(jnp.bfloat16),
                                  jnp.uint16).astype(jnp.uint32)
    x16_ref[...] = lax.bitcast_convert_type(xa | (xb << 16), jnp.int32)
    logits = jnp.dot(x, gw_ref[...],
                     preferred_element_type=jnp.float32) + gb_ref[...]
    r, e = logits.shape
    col = lax.broadcasted_iota(jnp.int32, (r, e), 1)
    m1 = jnp.max(logits, axis=1, keepdims=True)
    i1 = jnp.min(jnp.where(logits == m1, col, e), axis=1, keepdims=True)
    masked = jnp.where(col == i1, -jnp.inf, logits)
    m2 = jnp.max(masked, axis=1, keepdims=True)
    i2 = jnp.min(jnp.where(masked == m2, col, e), axis=1, keepdims=True)
    e2 = jnp.exp(m2 - m1)
    denom = 1.0 + e2
    i0_ref[...] = i1
    i1_ref[...] = i2
    s0_ref[...] = 1.0 / denom
    s1_ref[...] = e2 / denom


def _gate_call(moe_inp, gate_w, gate_b):
    n, d = moe_inp.shape
    e = gate_w.shape[1]
    r = _GATE_ROWS
    outs = pl.pallas_call(
        _gate_body,
        grid=(n // r,),
        in_specs=[
            pl.BlockSpec((r, d), lambda i: (i, 0)),
            pl.BlockSpec((d, e), lambda i: (0, 0)),
            pl.BlockSpec((1, e), lambda i: (0, 0)),
        ],
        out_specs=[pl.BlockSpec((r, 1), lambda i: (i, 0))] * 4
        + [pl.BlockSpec((r, d // 2), lambda i: (i, 0))],
        out_shape=[
            jax.ShapeDtypeStruct((n, 1), jnp.int32),
            jax.ShapeDtypeStruct((n, 1), jnp.int32),
            jax.ShapeDtypeStruct((n, 1), jnp.float32),
            jax.ShapeDtypeStruct((n, 1), jnp.float32),
            jax.ShapeDtypeStruct((n, d // 2), jnp.int32),
        ],
    )(moe_inp, gate_w, gate_b.reshape(1, e))
    return outs


def _make_meta_body(n_exp, npad):
    t = _TILE
    nt = npad // t

    def _meta_body(ef_ref, dst_ref, te_ref):
        l, c = ef_ref.shape
        lane_e = lax.broadcasted_iota(jnp.int32, (l, n_exp), 1)
        r_io = lax.broadcasted_iota(jnp.int32, (l, l), 0)
        c_io = lax.broadcasted_iota(jnp.int32, (l, l), 1)
        tril_s = (r_io > c_io).astype(jnp.float32)

        counts = jnp.zeros((1, n_exp), jnp.float32)
        for j in range(c):
            oh = (ef_ref[:, j:j + 1] == lane_e).astype(jnp.float32)
            counts = counts + jnp.sum(oh, axis=0, keepdims=True)

        caps = jnp.floor((counts + (t - 1)) * (1.0 / t)) * t
        er = lax.broadcasted_iota(jnp.int32, (n_exp, n_exp), 0)
        ec = lax.broadcasted_iota(jnp.int32, (n_exp, n_exp), 1)
        sut = (er < ec).astype(jnp.float32)
        off = jnp.dot(caps, sut, preferred_element_type=jnp.float32)

        carry = jnp.zeros((1, n_exp), jnp.float32)
        for j in range(c):
            oh = (ef_ref[:, j:j + 1] == lane_e).astype(jnp.float32)
            r_c = jnp.dot(tril_s, oh, preferred_element_type=jnp.float32)
            dstv = jnp.sum(oh * (off + carry + r_c), axis=1, keepdims=True)
            dst_ref[:, j:j + 1] = dstv.astype(jnp.int32)
            carry = carry + jnp.sum(oh, axis=0, keepdims=True)

        lane_idx = lax.broadcasted_iota(jnp.int32, (1, n_exp), 1)
        end = jnp.where(lane_idx == n_exp - 1, float(npad), off + caps)
        tstart = lax.broadcasted_iota(
            jnp.int32, (nt, n_exp), 0).astype(jnp.float32) * t
        ind = (tstart >= off) & (tstart < end)
        te_f = lax.broadcasted_iota(
            jnp.int32, (nt, n_exp), 1).astype(jnp.float32)
        te = jnp.sum(jnp.where(ind, te_f, 0.0), axis=1, keepdims=True)
        te_ref[...] = te.astype(jnp.int32)

    return _meta_body


def _meta_call(ef_t, n_exp, npad):
    l, c = ef_t.shape
    nt = npad // _TILE
    return pl.pallas_call(
        _make_meta_body(n_exp, npad),
        out_shape=(
            jax.ShapeDtypeStruct((l, c), jnp.int32),
            jax.ShapeDtypeStruct((nt, 1), jnp.int32),
        ),
    )(ef_t)


def _sc_mesh():
    return plsc.VectorSubcoreMesh(core_axis_name="c", subcore_axis_name="s")


def _make_scatter(n, dpack, npad):
    per_w = n // _NWORK
    nch = per_w // _CHUNK

    @functools.partial(
        pl.kernel,
        mesh=_sc_mesh(),
        out_type=(
            jax.ShapeDtypeStruct((npad, dpack), jnp.int32),
            jax.ShapeDtypeStruct((npad,), jnp.float32),
        ),
        scratch_types=(
            pltpu.VMEM((2, _CHUNK, dpack), jnp.int32),
            pltpu.VMEM((2, _CHUNK), jnp.int32),
            pltpu.VMEM((2, _CHUNK), jnp.int32),
            pltpu.VMEM((2, _CHUNK), jnp.float32),
            pltpu.VMEM((2, _CHUNK), jnp.float32),
            pltpu.SemaphoreType.DMA,
            pltpu.SemaphoreType.DMA,
            pltpu.SemaphoreType.DMA,
            pltpu.SemaphoreType.DMA,
        ),
    )
    def scat(moe_hbm, d0_hbm, d1_hbm, s0_hbm, s1_hbm, xs_hbm, ws_hbm,
             rows_v, i0_v, i1_v, v0_v, v1_v, ld0, ld1, st0, st1):
        wid = lax.axis_index("s") * 2 + lax.axis_index("c")
        lds = (ld0, ld1)
        sts = (st0, st1)
        ld_descs = [None, None]
        st_descs = [None, None]

        def load(ch):
            p = ch & 1
            base = wid * per_w + ch * _CHUNK
            ld_descs[p] = [
                pltpu.async_copy(moe_hbm.at[pl.ds(base, _CHUNK)],
                                 rows_v.at[p], lds[p]),
                pltpu.async_copy(d0_hbm.at[pl.ds(base, _CHUNK)],
                                 i0_v.at[p], lds[p]),
                pltpu.async_copy(d1_hbm.at[pl.ds(base, _CHUNK)],
                                 i1_v.at[p], lds[p]),
                pltpu.async_copy(s0_hbm.at[pl.ds(base, _CHUNK)],
                                 v0_v.at[p], lds[p]),
                pltpu.async_copy(s1_hbm.at[pl.ds(base, _CHUNK)],
                                 v1_v.at[p], lds[p]),
            ]

        def store(ch):
            p = ch & 1
            st_descs[p] = [
                pltpu.async_copy(rows_v.at[p], xs_hbm.at[i0_v.at[p]], sts[p]),
                pltpu.async_copy(rows_v.at[p], xs_hbm.at[i1_v.at[p]], sts[p]),
                pltpu.async_copy(v0_v.at[p], ws_hbm.at[i0_v.at[p]], sts[p]),
                pltpu.async_copy(v1_v.at[p], ws_hbm.at[i1_v.at[p]], sts[p]),
            ]

        load(0)
        for ch in range(nch):
            p = ch & 1
            if ch + 1 < nch:
                if ch >= 1:
                    for dd in st_descs[1 - p]:
                        dd.wait()
                load(ch + 1)
            for dd in ld_descs[p]:
                dd.wait()
            store(ch)
        for dd in st_descs[(nch - 2) & 1]:
            dd.wait()
        for dd in st_descs[(nch - 1) & 1]:
            dd.wait()

    return scat


def _ffn_body(te_ref, x_ref, w1_ref, b1_ref, w2_ref, b2_ref, ws_ref, y_ref):
    del te_ref
    xi = x_ref[...]
    lo = lax.bitcast_convert_type(xi << 16, jnp.float32)
    hi = lax.bitcast_convert_type(xi & jnp.int32(-65536), jnp.float32)
    x = jnp.concatenate([lo, hi], axis=1)
    h = jnp.dot(x, w1_ref[0],
                preferred_element_type=jnp.float32) + b1_ref[0]
    h = jnp.maximum(h, 0.0)
    y = jnp.dot(h, w2_ref[0], preferred_element_type=jnp.float32) + b2_ref[0]
    y = y * ws_ref[...]
    half = y.shape[1] // 2
    ya = lax.bitcast_convert_type(y[:, :half].astype(jnp.bfloat16),
                                  jnp.uint16).astype(jnp.uint32)
    yb = lax.bitcast_convert_type(y[:, half:].astype(jnp.bfloat16),
                                  jnp.uint16).astype(jnp.uint32)
    y_ref[...] = lax.bitcast_convert_type(ya | (yb << 16), jnp.int32)


def _ffn_call(te, xs, w1, b1, w2, b2, ws, npad):
    e, d, _ = w1.shape
    t = _TILE
    nt = npad // t
    grid_spec = pltpu.PrefetchScalarGridSpec(
        num_scalar_prefetch=1,
        grid=(nt,),
        in_specs=[
            pl.BlockSpec((t, d // 2), lambda i, te_s: (i, 0)),
            pl.BlockSpec((1, d, d), lambda i, te_s: (te_s[i], 0, 0)),
            pl.BlockSpec((1, 1, d), lambda i, te_s: (te_s[i], 0, 0)),
            pl.BlockSpec((1, d, d), lambda i, te_s: (te_s[i], 0, 0)),
            pl.BlockSpec((1, 1, d), lambda i, te_s: (te_s[i], 0, 0)),
            pl.BlockSpec((t, 1), lambda i, te_s: (i, 0)),
        ],
        out_specs=pl.BlockSpec((t, d // 2), lambda i, te_s: (i, 0)),
    )
    return pl.pallas_call(
        _ffn_body,
        grid_spec=grid_spec,
        out_shape=jax.ShapeDtypeStruct((npad, d // 2), jnp.int32),
    )(te, xs, w1, b1.reshape(e, 1, d), w2, b2.reshape(e, 1, d),
      ws.reshape(npad, 1))


def _make_combine(n, d, npad):
    cch = 32
    per_w = n // _NWORK
    nch = per_w // cch
    dpack = d // 2
    nvec = dpack // 16

    @functools.partial(
        pl.kernel,
        mesh=_sc_mesh(),
        out_type=jax.ShapeDtypeStruct((n, d), jnp.float32),
        scratch_types=(
            pltpu.VMEM((2, cch, dpack), jnp.int32),
            pltpu.VMEM((2, cch, dpack), jnp.int32),
            pltpu.VMEM((2, cch, d), jnp.float32),
            pltpu.VMEM((2, cch), jnp.int32),
            pltpu.VMEM((2, cch), jnp.int32),
            pltpu.SemaphoreType.DMA,
            pltpu.SemaphoreType.DMA,
            pltpu.SemaphoreType.DMA,
            pltpu.SemaphoreType.DMA,
            pltpu.SemaphoreType.DMA,
            pltpu.SemaphoreType.DMA,
        ),
    )
    def comb(y_hbm, d0_hbm, d1_hbm, out_hbm, b0_v, b1_v, o_v, i0_v, i1_v,
             li0, li1, gd0, gd1, st0, st1):
        wid = lax.axis_index("s") * 2 + lax.axis_index("c")
        lis = (li0, li1)
        gds = (gd0, gd1)
        sts = (st0, st1)
        li_descs = [None, None]
        gd_descs = [None, None]
        st_descs = [None, None]

        def loadidx(ch):
            p = ch & 1
            base = wid * per_w + ch * cch
            li_descs[p] = [
                pltpu.async_copy(d0_hbm.at[pl.ds(base, cch)],
                                 i0_v.at[p], lis[p]),
                pltpu.async_copy(d1_hbm.at[pl.ds(base, cch)],
                                 i1_v.at[p], lis[p]),
            ]

        def gather(ch):
            p = ch & 1
            for dd in li_descs[p]:
                dd.wait()
            gd_descs[p] = [
                pltpu.async_copy(y_hbm.at[i0_v.at[p]], b0_v.at[p], gds[p]),
                pltpu.async_copy(y_hbm.at[i1_v.at[p]], b1_v.at[p], gds[p]),
            ]

        loadidx(0)
        gather(0)
        for ch in range(nch):
            p = ch & 1
            base = wid * per_w + ch * cch
            if ch + 1 < nch:
                loadidx(ch + 1)
            for dd in gd_descs[p]:
                dd.wait()
            if ch + 1 < nch:
                if ch >= 1:
                    for dd in st_descs[1 - p]:
                        dd.wait()
                gather(ch + 1)

            @plsc.parallel_loop(0, cch * nvec, unroll=8)
            def _add(idx):
                r = idx // nvec
                col = (idx % nvec) * 16
                v0 = b0_v[p, r, pl.ds(col, 16)]
                v1 = b1_v[p, r, pl.ds(col, 16)]
                mask = jnp.int32(-65536)
                lo = (lax.bitcast_convert_type(v0 << 16, jnp.float32)
                      + lax.bitcast_convert_type(v1 << 16, jnp.float32))
                hi = (lax.bitcast_convert_type(v0 & mask, jnp.float32)
                      + lax.bitcast_convert_type(v1 & mask, jnp.float32))
                o_v[p, r, pl.ds(col, 16)] = lo
                o_v[p, r, pl.ds(dpack + col, 16)] = hi

            st_descs[p] = [
                pltpu.async_copy(o_v.at[p],
                                 out_hbm.at[pl.ds(base, cch)], sts[p]),
            ]
        for dd in st_descs[(nch - 2) & 1]:
            dd.wait()
        for dd in st_descs[(nch - 1) & 1]:
            dd.wait()

    return comb


def kernel(moe_inp, gate_w, gate_b, w1, b1, w2, b2):
    n, d = moe_inp.shape
    n_exp = w1.shape[0]
    npad = 2 * n + n_exp * _TILE

    i0, i1, s0, s1, x16 = _gate_call(moe_inp, gate_w, gate_b)

    ef = jnp.concatenate([i0, i1], axis=0)
    ef_t = ef.reshape(_LSUB, (2 * n) // _LSUB)
    dst_lc, te2 = _meta_call(ef_t, n_exp, npad)
    dstf = dst_lc.reshape(-1)
    d0 = dstf[:n]
    d1 = dstf[n:]

    xs, ws = _make_scatter(n, d // 2, npad)(
        x16, d0, d1, s0.reshape(n), s1.reshape(n))

    y = _ffn_call(te2.reshape(-1), xs, w1, b1, w2, b2, ws, npad)

    out = _make_combine(n, d, npad)(y, d0, d1)
    return out
